# baseline (speedup 1.0000x reference)
import jax
import jax.numpy as jnp
from jax.experimental import pallas as pl
from jax.experimental.pallas import tpu as pltpu

H = 8
S = 256
T = 1024
D = 64
SZ_B = 16
B = SZ_B * H
RATIO = T // S
W = 2 * S

T_BLK = 128
G = T_BLK // RATIO
WIN = 384
W_PAD = (S - G) + WIN


def _rel_attn_kernel(q_ref, c_ref, o_ref):
    j = pl.program_id(2)
    base = (S - G) - G * j
    c_win = c_ref[0, pl.ds(base, WIN), :]
    sc = jax.lax.dot_general(
        q_ref[0], c_win,
        (((1,), (1,)), ((), ())),
        preferred_element_type=jnp.float32,
    )
    r = jax.lax.broadcasted_iota(jnp.int32, (T_BLK, 1), 0)
    resid = (G - 1) - r // RATIO
    x = sc
    for k in range(5):
        rolled = jnp.roll(x, -(1 << k), axis=1)
        x = jnp.where(((resid >> k) & 1) == 1, rolled, x)
    o_ref[0] = x[:, :S]


@jax.jit
def kernel(q, e1, e2):
    e1h = e1.reshape(H, S, D)
    e2h = e2.reshape(H, S, D)
    c = jnp.concatenate(
        [e1h, e2h[:, 1:, :], jnp.zeros((H, 1 + W_PAD - W, D), e2h.dtype)],
        axis=1)

    grid = (H, SZ_B, T // T_BLK)
    return pl.pallas_call(
        _rel_attn_kernel,
        grid=grid,
        in_specs=[
            pl.BlockSpec((1, T_BLK, D), lambda h, b, j: (b * H + h, j, 0)),
            pl.BlockSpec((1, W_PAD, D), lambda h, b, j: (h, 0, 0)),
        ],
        out_specs=pl.BlockSpec((1, T_BLK, S), lambda h, b, j: (b * H + h, j, 0)),
        out_shape=jax.ShapeDtypeStruct((B, T, S), jnp.float32),
        compiler_params=pltpu.CompilerParams(
            dimension_semantics=("parallel", "parallel", "arbitrary"),
        ),
    )(q, c)


# speedup vs baseline: 1.5397x; 1.5397x over previous
import jax
import jax.numpy as jnp
from jax.experimental import pallas as pl
from jax.experimental.pallas import tpu as pltpu

H = 8
S = 256
T = 1024
D = 64
SZ_B = 16
B = SZ_B * H
RATIO = T // S
W = 2 * S

T_BLK = 256
G = T_BLK // RATIO
WIN = 384
W_PAD = (S - G) + WIN


def _rel_attn_kernel(q_ref, c_ref, o_ref):
    j = pl.program_id(2)
    base = (S - G) - G * j
    c_win = c_ref[0, pl.ds(base, WIN), :]
    sc = jax.lax.dot_general(
        q_ref[0], c_win,
        (((1,), (1,)), ((), ())),
        preferred_element_type=jnp.float32,
    )
    r = jax.lax.broadcasted_iota(jnp.int32, (T_BLK, 1), 0)
    resid = (G - 1) - r // RATIO
    x = sc
    for k in range(6):
        rolled = jnp.roll(x, -(1 << k), axis=1)
        x = jnp.where(((resid >> k) & 1) == 1, rolled, x)
    o_ref[0] = x[:, :S]


@jax.jit
def kernel(q, e1, e2):
    e1h = e1.reshape(H, S, D)
    e2h = e2.reshape(H, S, D)
    c = jnp.concatenate(
        [e1h, e2h[:, 1:, :], jnp.zeros((H, 1 + W_PAD - W, D), e2h.dtype)],
        axis=1)

    grid = (H, SZ_B, T // T_BLK)
    return pl.pallas_call(
        _rel_attn_kernel,
        grid=grid,
        in_specs=[
            pl.BlockSpec((1, T_BLK, D), lambda h, b, j: (b * H + h, j, 0)),
            pl.BlockSpec((1, W_PAD, D), lambda h, b, j: (h, 0, 0)),
        ],
        out_specs=pl.BlockSpec((1, T_BLK, S), lambda h, b, j: (b * H + h, j, 0)),
        out_shape=jax.ShapeDtypeStruct((B, T, S), jnp.float32),
        compiler_params=pltpu.CompilerParams(
            dimension_semantics=("parallel", "parallel", "arbitrary"),
        ),
    )(q, c)


# speedup vs baseline: 1.8805x; 1.2213x over previous
import jax
import jax.numpy as jnp
from jax.experimental import pallas as pl
from jax.experimental.pallas import tpu as pltpu

H = 8
S = 256
T = 1024
D = 64
SZ_B = 16
B = SZ_B * H
RATIO = T // S
W = 2 * S

T_BLK = 256
G = T_BLK // RATIO
WIN = 384
W_PAD = (S - G) + WIN


def _rel_attn_kernel(q_ref, c_ref, o_ref):
    j = pl.program_id(2)
    base = (S - G) - G * j
    c_win = c_ref[0, pl.ds(base, WIN), :]
    sc = jax.lax.dot_general(
        q_ref[0], c_win,
        (((1,), (1,)), ((), ())),
        preferred_element_type=jnp.float32,
    )
    r = jax.lax.broadcasted_iota(jnp.int32, (T_BLK, 128), 0)
    s128 = jax.lax.broadcasted_iota(jnp.int32, (T_BLK, 128), 1)
    resid = (G - 1) - r // RATIO
    idxw = (s128 + resid) & 127
    cross = (s128 + resid) >= 128
    cols = []
    for c in range(S // 128):
        src_a = sc[:, c * 128:(c + 1) * 128]
        src_b = sc[:, (c + 1) * 128:(c + 2) * 128]
        g_a = jnp.take_along_axis(src_a, idxw, axis=1)
        g_b = jnp.take_along_axis(src_b, idxw, axis=1)
        cols.append(jnp.where(cross, g_b, g_a))
    o_ref[0] = jnp.concatenate(cols, axis=1)


@jax.jit
def kernel(q, e1, e2):
    e1h = e1.reshape(H, S, D)
    e2h = e2.reshape(H, S, D)
    c = jnp.concatenate(
        [e1h, e2h[:, 1:, :], jnp.zeros((H, 1 + W_PAD - W, D), e2h.dtype)],
        axis=1)

    grid = (H, SZ_B, T // T_BLK)
    return pl.pallas_call(
        _rel_attn_kernel,
        grid=grid,
        in_specs=[
            pl.BlockSpec((1, T_BLK, D), lambda h, b, j: (b * H + h, j, 0)),
            pl.BlockSpec((1, W_PAD, D), lambda h, b, j: (h, 0, 0)),
        ],
        out_specs=pl.BlockSpec((1, T_BLK, S), lambda h, b, j: (b * H + h, j, 0)),
        out_shape=jax.ShapeDtypeStruct((B, T, S), jnp.float32),
        compiler_params=pltpu.CompilerParams(
            dimension_semantics=("parallel", "parallel", "arbitrary"),
        ),
    )(q, c)


# speedup vs baseline: 2.7390x; 1.4565x over previous
import jax
import jax.numpy as jnp
from jax.experimental import pallas as pl
from jax.experimental.pallas import tpu as pltpu

H = 8
S = 256
T = 1024
D = 64
SZ_B = 16
B = SZ_B * H
RATIO = T // S
W = 2 * S

T_BLK = 512
G = T_BLK // RATIO
WIN = 384
W_PAD = (S - G) + WIN


def _rel_attn_kernel(q_ref, c_ref, o_ref):
    j = pl.program_id(2)
    base = (S - G) - G * j
    c_win = c_ref[0, pl.ds(base, WIN), :]
    sc = jax.lax.dot_general(
        q_ref[0], c_win,
        (((1,), (1,)), ((), ())),
        preferred_element_type=jnp.float32,
    )
    r = jax.lax.broadcasted_iota(jnp.int32, (T_BLK, 128), 0)
    s128 = jax.lax.broadcasted_iota(jnp.int32, (T_BLK, 128), 1)
    resid = (G - 1) - r // RATIO
    idxw = (s128 + resid) & 127
    cross = (s128 + resid) >= 128
    cols = []
    for c in range(S // 128):
        src_a = sc[:, c * 128:(c + 1) * 128]
        src_b = sc[:, (c + 1) * 128:(c + 2) * 128]
        g_a = jnp.take_along_axis(src_a, idxw, axis=1)
        g_b = jnp.take_along_axis(src_b, idxw, axis=1)
        cols.append(jnp.where(cross, g_b, g_a))
    o_ref[0] = jnp.concatenate(cols, axis=1)


@jax.jit
def kernel(q, e1, e2):
    e1h = e1.reshape(H, S, D)
    e2h = e2.reshape(H, S, D)
    c = jnp.concatenate(
        [e1h, e2h[:, 1:, :], jnp.zeros((H, 1 + W_PAD - W, D), e2h.dtype)],
        axis=1)

    grid = (H, SZ_B, T // T_BLK)
    return pl.pallas_call(
        _rel_attn_kernel,
        grid=grid,
        in_specs=[
            pl.BlockSpec((1, T_BLK, D), lambda h, b, j: (b * H + h, j, 0)),
            pl.BlockSpec((1, W_PAD, D), lambda h, b, j: (h, 0, 0)),
        ],
        out_specs=pl.BlockSpec((1, T_BLK, S), lambda h, b, j: (b * H + h, j, 0)),
        out_shape=jax.ShapeDtypeStruct((B, T, S), jnp.float32),
        compiler_params=pltpu.CompilerParams(
            dimension_semantics=("parallel", "parallel", "arbitrary"),
        ),
    )(q, c)
